# Initial kernel scaffold; baseline (speedup 1.0000x reference)
import jax
import jax.numpy as jnp
from jax.experimental import pallas as pl


def kernel(user_emb, item_emb, group_emb, uh_rows, uh_cols, uh_vals, ih_rows, ih_cols, ih_vals, fh_rows, fh_cols, fh_vals, W0, b0, W1, b1, num_users, num_items):
    raise NotImplementedError("write your pallas kernel here")


# speedup vs baseline: 3.3880x; 3.3880x over previous
import jax
import jax.numpy as jnp
from jax import lax
from jax.experimental import pallas as pl
from jax.experimental.pallas import tpu as pltpu
from jax.experimental.pallas import tpu_sc as plsc

NC = 2
NS = 16
LN = 16
CH = 128

_BCAST_DNUMS = lax.GatherDimensionNumbers(
    offset_dims=(), collapsed_slice_dims=(0,), start_index_map=(0,))


def _bcast_lane(vec, e):
    idx = jnp.full((LN, 1), e, jnp.int32)
    return lax.gather(vec, idx, _BCAST_DNUMS, (1,),
                      mode=lax.GatherScatterMode.PROMISE_IN_BOUNDS)


def _zero_buf(ref, nrows, d):
    z = jnp.zeros((LN,), jnp.float32)

    def body(r, carry):
        for k in range(d // LN):
            ref[r, pl.ds(k * LN, LN)] = z
        return carry

    lax.fori_loop(0, nrows, body, 0)


def _scale_rows(gbuf, valb, d, mask_fn=None, rowb=None, idxb=None):
    def grp(g, carry):
        vv = valb[pl.ds(g * LN, LN)]
        if mask_fn is not None:
            rv = rowb[pl.ds(g * LN, LN)]
            vv, lr = mask_fn(vv, rv)
            idxb[pl.ds(g * LN, LN)] = lr
        for e in range(LN):
            val = _bcast_lane(vv, e)
            r = g * LN + e
            for k in range(d // LN):
                gbuf[r, pl.ds(k * LN, LN)] = gbuf[r, pl.ds(k * LN, LN)] * val
        return carry

    lax.fori_loop(0, CH // LN, grp, 0)


def _row_partition(n):
    rps8 = -(-(-(-n // NS)) // 8) * 8
    last = n - (NS - 1) * rps8
    assert last > 0 and last % 8 == 0 and n % 8 == 0
    return rps8, last


def _zero_acc(s, zbuf, acc, base_off, rps8, last):
    for cnt, pred in ((rps8, s < NS - 1), (last, s == NS - 1)):
        nfull, nrem = divmod(cnt, CH)

        @pl.when(pred)
        def _():
            base = base_off + s * rps8
            for j in range(nfull):
                pltpu.sync_copy(zbuf, acc.at[pl.ds(base + j * CH, CH)])
            if nrem:
                pltpu.sync_copy(zbuf.at[pl.ds(0, nrem)],
                                acc.at[pl.ds(base + nfull * CH, nrem)])


def _writeout(s, acc, out, out_off, rps8, last):
    for cnt, pred in ((rps8, s < NS - 1), (last, s == NS - 1)):

        @pl.when(pred)
        def _():
            pltpu.sync_copy(acc.at[pl.ds(s * rps8, cnt)],
                            out.at[pl.ds(out_off + s * rps8, cnt)])


def _phase_a_body(G, D, ea_pad):
    nch = ea_pad // CH
    rps8, last = _row_partition(G)

    def body(table, cols, rows, vals, out, zbuf, gbuf, colb, rowb, valb,
             acc, sem):
        c = lax.axis_index("c")
        s = lax.axis_index("s")
        _zero_buf(zbuf, CH, D)
        _zero_acc(s, zbuf, acc, 0, rps8, last)
        plsc.subcore_barrier()

        e0 = (c * NS + s) * ea_pad

        def chunk(i, carry):
            eb = e0 + i * CH
            pltpu.sync_copy(cols.at[pl.ds(eb, CH)], colb)
            pltpu.sync_copy(rows.at[pl.ds(eb, CH)], rowb)
            pltpu.sync_copy(vals.at[pl.ds(eb, CH)], valb)
            pltpu.async_copy(table.at[colb], gbuf, sem).wait()
            _scale_rows(gbuf, valb, D)
            pltpu.sync_copy(gbuf, acc.at[rowb], add=True)
            return carry

        lax.fori_loop(0, nch, chunk, 0)
        plsc.subcore_barrier()
        _writeout(s, acc, out, c * G, rps8, last)

    return body


def _phase_c_body(G, D, n_out, tile_rows):
    n_tiles = n_out // tile_rows
    tpc = n_tiles // NC
    rps8, last = _row_partition(tile_rows)

    def body(msg, cols, rows, vals, ptr, out, zbuf, gbuf, colb, rowb, valb,
             idxb, ptrb, acc, sem):
        c = lax.axis_index("c")
        s = lax.axis_index("s")
        _zero_buf(zbuf, CH, D)
        pltpu.sync_copy(ptr, ptrb)
        for tl in range(tpc):
            t = c * tpc + tl
            pv = ptrb[pl.ds(t, LN)]
            lo = pv[0]
            hi = pv[1]
            lo_al = (lo // 8) * 8
            total_ch = (hi - lo_al + CH - 1) // CH
            tile_lo = t * tile_rows
            _zero_acc(s, zbuf, acc, 0, rps8, last)
            plsc.subcore_barrier()

            def mask_fn(vv, rv):
                ok = (rv >= tile_lo) & (rv < tile_lo + tile_rows)
                vv = jnp.where(ok, vv, 0.0)
                lr = jnp.clip(rv - tile_lo, 0, tile_rows - 1)
                return vv, lr

            def chunk(i, carry):
                eb = lo_al + (s + i * NS) * CH
                pltpu.sync_copy(cols.at[pl.ds(eb, CH)], colb)
                pltpu.sync_copy(rows.at[pl.ds(eb, CH)], rowb)
                pltpu.sync_copy(vals.at[pl.ds(eb, CH)], valb)
                pltpu.async_copy(msg.at[colb], gbuf, sem).wait()
                _scale_rows(gbuf, valb, D, mask_fn, rowb, idxb)
                pltpu.sync_copy(gbuf, acc.at[idxb], add=True)
                return carry

            my_nch = jnp.maximum((total_ch - s + NS - 1) // NS, 0)
            lax.fori_loop(0, my_nch, chunk, 0)
            plsc.subcore_barrier()
            _writeout(s, acc, out, tile_lo, rps8, last)
            plsc.subcore_barrier()

    return body


def _padto(x, n, fill):
    m = x.shape[0]
    if m == n:
        return x
    return jnp.concatenate([x, jnp.full((n - m,), fill, x.dtype)])


def kernel(user_emb, item_emb, group_emb, uh_rows, uh_cols, uh_vals,
           ih_rows, ih_cols, ih_vals, fh_rows, fh_cols, fh_vals,
           W0, b0, W1, b1, num_users, num_items):
    f32 = jnp.float32
    U, D = user_emb.shape
    I = item_emb.shape[0]
    G = group_emb.shape[0]
    N = U + I

    emb0 = jnp.concatenate([user_emb, item_emb], axis=0)

    nnz_a = max(uh_rows.shape[0], ih_rows.shape[0])
    ea_pad = -(-(-(-nnz_a // NS)) // CH) * CH
    apad = NS * ea_pad
    cols_a = jnp.concatenate([_padto(uh_cols, apad, 0),
                              _padto(ih_cols + U, apad, 0)])
    rows_a = jnp.concatenate([_padto(uh_rows, apad, 0),
                              _padto(ih_rows, apad, 0)])
    vals_a = jnp.concatenate([_padto(uh_vals, apad, 0.0),
                              _padto(ih_vals, apad, 0.0)])

    nnz_f = fh_rows.shape[0]
    fcols = _padto(fh_cols, nnz_f + CH, 0)
    frows = _padto(fh_rows, nnz_f + CH, 0)
    fvals = _padto(fh_vals, nnz_f + CH, 0.0)
    TILE = 10000
    n_tiles = N // TILE
    ptr = jnp.searchsorted(
        fh_rows, jnp.arange(n_tiles + 1, dtype=jnp.int32) * TILE).astype(jnp.int32)
    ptr32 = _padto(ptr, 2 * LN, nnz_f)

    mesh = plsc.VectorSubcoreMesh(core_axis_name="c", subcore_axis_name="s",
                                  num_cores=NC, num_subcores=NS)

    phase_a = pl.kernel(
        _phase_a_body(G, D, ea_pad),
        out_type=jax.ShapeDtypeStruct((NC * G, D), f32),
        mesh=mesh,
        scratch_types=[
            pltpu.VMEM((CH, D), f32),
            pltpu.VMEM((CH, D), f32),
            pltpu.VMEM((CH,), jnp.int32),
            pltpu.VMEM((CH,), jnp.int32),
            pltpu.VMEM((CH,), f32),
            pltpu.VMEM_SHARED((G, D), f32),
            pltpu.SemaphoreType.DMA,
        ],
    )

    phase_c = pl.kernel(
        _phase_c_body(G, D, N, TILE),
        out_type=jax.ShapeDtypeStruct((N, D), f32),
        mesh=mesh,
        scratch_types=[
            pltpu.VMEM((CH, D), f32),
            pltpu.VMEM((CH, D), f32),
            pltpu.VMEM((CH,), jnp.int32),
            pltpu.VMEM((CH,), jnp.int32),
            pltpu.VMEM((CH,), f32),
            pltpu.VMEM((CH,), jnp.int32),
            pltpu.VMEM((2 * LN,), jnp.int32),
            pltpu.VMEM_SHARED((TILE, D), f32),
            pltpu.SemaphoreType.DMA,
        ],
    )

    BLK = 2000

    def _combine(um_ref, im_ref, g_ref, W_ref, b_ref, he_ref, msg_ref, heo_ref):
        um = um_ref[...]
        im = im_ref[...]
        gg = g_ref[...]
        W = W_ref[...]
        m = (jnp.dot(um, W[0:D], preferred_element_type=f32)
             + jnp.dot(im, W[D:2 * D], preferred_element_type=f32)
             + jnp.dot(im * gg, W[2 * D:3 * D], preferred_element_type=f32)
             + b_ref[...])
        msg_ref[...] = m
        heo_ref[...] = he_ref[...] + m

    combine = pl.pallas_call(
        _combine,
        grid=(G // BLK,),
        in_specs=[pl.BlockSpec((BLK, D), lambda i: (i, 0))] * 3
        + [pl.BlockSpec((3 * D, D), lambda i: (0, 0)),
           pl.BlockSpec((1, D), lambda i: (0, 0)),
           pl.BlockSpec((BLK, D), lambda i: (i, 0))],
        out_specs=[pl.BlockSpec((BLK, D), lambda i: (i, 0))] * 2,
        out_shape=[jax.ShapeDtypeStruct((G, D), f32)] * 2,
    )

    BLK3 = 4000

    def _sum3(a_ref, b_ref, c_ref, o_ref):
        o_ref[...] = a_ref[...] + b_ref[...] + c_ref[...]

    sum3 = pl.pallas_call(
        _sum3,
        grid=(N // BLK3,),
        in_specs=[pl.BlockSpec((BLK3, D), lambda i: (i, 0))] * 3,
        out_specs=pl.BlockSpec((BLK3, D), lambda i: (i, 0)),
        out_shape=jax.ShapeDtypeStruct((N, D), f32),
    )

    b0r = b0.reshape(1, D)
    b1r = b1.reshape(1, D)

    msgs1 = phase_a(emb0, cols_a, rows_a, vals_a)
    msg1, he1 = combine(msgs1[:G], msgs1[G:], group_emb, W0, b0r, group_emb)
    emb1 = phase_c(msg1, fcols, frows, fvals, ptr32)
    msgs2 = phase_a(emb1, cols_a, rows_a, vals_a)
    msg2, he2 = combine(msgs2[:G], msgs2[G:], group_emb, W1, b1r, he1)
    emb2 = phase_c(msg2, fcols, frows, fvals, ptr32)

    final_emb = sum3(emb0, emb1, emb2)
    return (final_emb, he2)
